# baseline (speedup 1.0000x reference)
import jax
import jax.numpy as jnp
from jax import lax
from jax.experimental import pallas as pl
from jax.experimental.pallas import tpu as pltpu
from jax.experimental.pallas import tpu_sc as plsc

N_I = 4096
N_J = 50
B = N_I * N_J
D = 64
V = 100000
NC, NS = 2, 16
NW = NC * NS
D_PER_W = D // NW
L = 16


def _vgather_body(idx_hbm, pet_hbm, out_hbm, table_v, idx_v, out_v, sem_t,
                  sem_i, sem_o):
    wid = lax.axis_index("s") * NC + lax.axis_index("c")
    for t in range(D_PER_W):
        d = wid * D_PER_W + t
        pltpu.sync_copy(pet_hbm.at[pl.ds(d * V, V)], table_v)

        @pl.loop(0, N_J)
        def _j_loop(j):
            pltpu.sync_copy(idx_hbm.at[pl.ds(j * N_I, N_I)], idx_v)

            @pl.loop(0, N_I, step=L, unroll=8)
            def _v_loop(v):
                iv = idx_v[pl.ds(v, L)]
                out_v[pl.ds(v, L)] = plsc.load_gather(table_v, [iv])

            pltpu.sync_copy(
                out_v, out_hbm.at[pl.ds((j * D + d) * N_I, N_I)])


@jax.jit
def _sc_vgather(idx_flatT, pe_flatT):
    mesh = plsc.VectorSubcoreMesh(core_axis_name="c", subcore_axis_name="s")
    k = pl.kernel(
        _vgather_body,
        out_type=jax.ShapeDtypeStruct((N_J * D * N_I,), jnp.float32),
        mesh=mesh,
        scratch_types=[
            pltpu.VMEM((V,), jnp.float32),
            pltpu.VMEM((N_I,), jnp.int32),
            pltpu.VMEM((N_I,), jnp.float32),
            pltpu.SemaphoreType.DMA,
            pltpu.SemaphoreType.DMA,
            pltpu.SemaphoreType.DMA,
        ],
        compiler_params=pltpu.CompilerParams(needs_layout_passes=False),
    )
    return k(idx_flatT, pe_flatT)


def kernel(idxes, pe):
    idx_flatT = idxes.T.reshape(B).astype(jnp.int32)
    pe_flatT = pe.T.reshape(D * V)
    out_flat = _sc_vgather(idx_flatT, pe_flatT)
    return out_flat.reshape(N_J, D, N_I).transpose(2, 0, 1)


# speedup vs baseline: 2.2449x; 2.2449x over previous
import jax
import jax.numpy as jnp
from jax import lax
from jax.experimental import pallas as pl
from jax.experimental.pallas import tpu as pltpu
from jax.experimental.pallas import tpu_sc as plsc

N_I = 4096
N_J = 50
N_JP = 56
B = N_I * N_J
D = 64
DP = 128
V = 100000
NC, NS = 2, 16
NW = NC * NS
I_PER_W = N_I // NW
BLKS = 8
CHUNK = BLKS * N_J
NCHUNK = I_PER_W // BLKS
B_PER_W = I_PER_W * N_J
ROWS_V = CHUNK + (N_JP - N_J)


def _gather_body(idx_hbm, table_hbm, out_hbm, idx_v, rows0, rows1,
                 sem_g0, sem_g1, sem_o0, sem_o1):
    wid = lax.axis_index("s") * NC + lax.axis_index("c")
    base = wid * B_PER_W
    i_base = wid * I_PER_W
    rows = (rows0, rows1)
    sem_g = (sem_g0, sem_g1)
    sem_o = (sem_o0, sem_o1)
    pltpu.sync_copy(idx_hbm.at[pl.ds(base, B_PER_W)], idx_v)
    g = [None, None]
    o = [[], []]

    def writeback(c, buf):
        i0 = i_base + c * BLKS
        for k in range(BLKS):
            o[buf].append(pltpu.async_copy(
                rows[buf].at[pl.ds(k * N_J, N_JP)],
                out_hbm.at[i0 + k], sem_o[buf]))

    for c in range(NCHUNK):
        b = c % 2
        if c >= 2:
            for cp in o[b]:
                cp.wait()
            o[b] = []
        g[b] = pltpu.async_copy(
            table_hbm.at[idx_v.at[pl.ds(c * CHUNK, CHUNK)]],
            rows[b].at[pl.ds(0, CHUNK)], sem_g[b])
        if c >= 1:
            g[1 - b].wait()
            writeback(c - 1, 1 - b)
    lb = (NCHUNK - 1) % 2
    g[lb].wait()
    writeback(NCHUNK - 1, lb)
    for buf in (0, 1):
        for cp in o[buf]:
            cp.wait()


@jax.jit
def _sc_gather(idx_flat, pe_pad):
    mesh = plsc.VectorSubcoreMesh(core_axis_name="c", subcore_axis_name="s")
    k = pl.kernel(
        _gather_body,
        out_type=jax.ShapeDtypeStruct((N_I, N_JP, DP), jnp.float32),
        mesh=mesh,
        scratch_types=[
            pltpu.VMEM((B_PER_W,), jnp.int32),
            pltpu.VMEM((ROWS_V, DP), jnp.float32),
            pltpu.VMEM((ROWS_V, DP), jnp.float32),
            pltpu.SemaphoreType.DMA,
            pltpu.SemaphoreType.DMA,
            pltpu.SemaphoreType.DMA,
            pltpu.SemaphoreType.DMA,
        ],
    )
    return k(idx_flat, pe_pad)


def kernel(idxes, pe):
    idx_flat = jnp.minimum(idxes.reshape(B).astype(jnp.int32), V - 1)
    pe_pad = jnp.pad(pe, ((0, 0), (0, DP - D)))
    out56 = _sc_gather(idx_flat, pe_pad)
    return out56[:, :N_J, :D]
